# baseline (speedup 1.0000x reference)
import jax
import jax.numpy as jnp
from jax.experimental import pallas as pl
from jax.experimental.pallas import tpu as pltpu

_B, _N, _D, _H, _K = 16, 576, 384, 384, 5


def _dge_kernel(xc_ref, xp_ref, w1_ref, b1_ref, w2_ref, b2_ref, o_ref,
                s_ref):
    i = pl.program_id(0)
    cur = jax.lax.rem(i, 2)
    prv = jax.lax.rem(i + 1, 2)

    x = xc_ref[0]
    norm = jnp.sqrt(jnp.sum(x * x, axis=1, keepdims=True))
    xn = x / (norm + 1e-8)
    s = jax.lax.dot_general(
        xn, xn, (((1,), (1,)), ((), ())), preferred_element_type=jnp.float32
    )
    row = jax.lax.broadcasted_iota(jnp.int32, (_N, _N), 0)
    col = jax.lax.broadcasted_iota(jnp.int32, (_N, _N), 1)
    neg_inf = jnp.float32(-jnp.inf)
    s_ref[cur] = jnp.where(row == col, neg_inf, s)

    sp = s_ref[prv]
    xprev = xp_ref[0]
    v1 = jnp.max(sp, axis=1, keepdims=True)
    m = v1
    for _ in range(_K - 1):
        m = jnp.max(jnp.where(sp < m, sp, neg_inf), axis=1, keepdims=True)

    e = jnp.exp(sp - v1)
    p = jnp.where(sp >= m, e, 0.0)
    denom = jnp.sum(p, axis=1, keepdims=True)
    agg = jnp.dot(p, xprev, preferred_element_type=jnp.float32) / denom
    h = xprev + agg
    h = jax.lax.dot_general(
        h, w1_ref[...], (((1,), (1,)), ((), ())),
        preferred_element_type=jnp.float32,
    )
    h = jnp.maximum(h + b1_ref[...], 0.0)
    h = jax.lax.dot_general(
        h, w2_ref[...], (((1,), (1,)), ((), ())),
        preferred_element_type=jnp.float32,
    )
    o_ref[0] = jnp.maximum(h + b2_ref[...], 0.0)


def kernel(x, W1, b1, W2, b2):
    b1r = b1.reshape(1, _H)
    b2r = b2.reshape(1, _H)
    out = pl.pallas_call(
        _dge_kernel,
        grid=(_B + 1,),
        in_specs=[
            pl.BlockSpec((1, _N, _D), lambda i: (jnp.minimum(i, _B - 1), 0, 0)),
            pl.BlockSpec((1, _N, _D), lambda i: (jnp.maximum(i - 1, 0), 0, 0)),
            pl.BlockSpec((_H, _D), lambda i: (0, 0)),
            pl.BlockSpec((1, _H), lambda i: (0, 0)),
            pl.BlockSpec((_H, _H), lambda i: (0, 0)),
            pl.BlockSpec((1, _H), lambda i: (0, 0)),
        ],
        out_specs=pl.BlockSpec((1, _N, _H), lambda i: (jnp.maximum(i - 1, 0), 0, 0)),
        out_shape=jax.ShapeDtypeStruct((_B, _N, _H), jnp.float32),
        scratch_shapes=[pltpu.VMEM((2, _N, _N), jnp.float32)],
    )(x, x, W1, b1r, W2, b2r)
    return out


# speedup vs baseline: 1.1270x; 1.1270x over previous
import jax
import jax.numpy as jnp
from jax.experimental import pallas as pl
from jax.experimental.pallas import tpu as pltpu

_B, _N, _D, _H, _K = 16, 576, 384, 384, 5


def _dge_kernel(x_ref, w1_ref, b1_ref, w2_ref, b2_ref, o_ref):
    x = x_ref[0]
    norm = jnp.sqrt(jnp.sum(x * x, axis=1, keepdims=True))
    xn = x / (norm + 1e-8)
    s = jax.lax.dot_general(
        xn, xn, (((1,), (1,)), ((), ())), preferred_element_type=jnp.float32
    )
    row = jax.lax.broadcasted_iota(jnp.int32, (_N, _N), 0)
    col = jax.lax.broadcasted_iota(jnp.int32, (_N, _N), 1)
    neg_inf = jnp.float32(-jnp.inf)
    s = jnp.where(row == col, neg_inf, s)

    v1 = jnp.max(s, axis=1, keepdims=True)
    m = v1
    for _ in range(_K - 1):
        m = jnp.max(jnp.where(s < m, s, neg_inf), axis=1, keepdims=True)

    e = jnp.exp(s - v1)
    p = jnp.where(s >= m, e, 0.0)
    denom = jnp.sum(p, axis=1, keepdims=True)
    agg = jnp.dot(p, x, preferred_element_type=jnp.float32) / denom
    h = x + agg
    h = jax.lax.dot_general(
        h, w1_ref[...], (((1,), (1,)), ((), ())),
        preferred_element_type=jnp.float32,
    )
    h = jnp.maximum(h + b1_ref[...], 0.0)
    h = jax.lax.dot_general(
        h, w2_ref[...], (((1,), (1,)), ((), ())),
        preferred_element_type=jnp.float32,
    )
    o_ref[0] = jnp.maximum(h + b2_ref[...], 0.0)


def kernel(x, W1, b1, W2, b2):
    b1r = b1.reshape(1, _H)
    b2r = b2.reshape(1, _H)
    out = pl.pallas_call(
        _dge_kernel,
        grid=(_B,),
        in_specs=[
            pl.BlockSpec((1, _N, _D), lambda b: (b, 0, 0)),
            pl.BlockSpec((_H, _D), lambda b: (0, 0)),
            pl.BlockSpec((1, _H), lambda b: (0, 0)),
            pl.BlockSpec((_H, _H), lambda b: (0, 0)),
            pl.BlockSpec((1, _H), lambda b: (0, 0)),
        ],
        out_specs=pl.BlockSpec((1, _N, _H), lambda b: (b, 0, 0)),
        out_shape=jax.ShapeDtypeStruct((_B, _N, _H), jnp.float32),
        compiler_params=pltpu.CompilerParams(
            dimension_semantics=("parallel",),
        ),
    )(x, W1, b1r, W2, b2r)
    return out
